# baseline (speedup 1.0000x reference)
import functools
import jax
import jax.numpy as jnp
from jax.experimental import pallas as pl
from jax.experimental.pallas import tpu as pltpu

N_NODES = 10000
N_EDGES = 320000
DIM = 128
HEADS = 3


def _ab_body(h_src_ref, h_dst_ref, qwt_ref, qb_ref, w1_ref, w2_ref,
             z_ref, s1_ref, t2_ref):
    z = jnp.maximum(jnp.dot(h_src_ref[...], qwt_ref[...],
                            preferred_element_type=jnp.float32)
                    + qb_ref[...], 0.0)
    z_ref[...] = z
    s1_ref[...] = jnp.dot(z, w1_ref[...], preferred_element_type=jnp.float32)
    t2_ref[...] = jnp.dot(h_dst_ref[...], w2_ref[...],
                          preferred_element_type=jnp.float32)


def _node_precompute(h_src, h_dst, QwT, Qb, W1p, W2p):
    BN = 400
    grid = (N_NODES // BN,)
    return pl.pallas_call(
        _ab_body,
        grid=grid,
        in_specs=[
            pl.BlockSpec((BN, DIM), lambda i: (i, 0)),
            pl.BlockSpec((BN, DIM), lambda i: (i, 0)),
            pl.BlockSpec((DIM, DIM), lambda i: (0, 0)),
            pl.BlockSpec((1, DIM), lambda i: (0, 0)),
            pl.BlockSpec((DIM, 8), lambda i: (0, 0)),
            pl.BlockSpec((DIM, 8), lambda i: (0, 0)),
        ],
        out_specs=[
            pl.BlockSpec((BN, DIM), lambda i: (i, 0)),
            pl.BlockSpec((BN, 8), lambda i: (i, 0)),
            pl.BlockSpec((BN, 8), lambda i: (i, 0)),
        ],
        out_shape=[
            jax.ShapeDtypeStruct((N_NODES, DIM), jnp.float32),
            jax.ShapeDtypeStruct((N_NODES, 8), jnp.float32),
            jax.ShapeDtypeStruct((N_NODES, 8), jnp.float32),
        ],
    )(h_src, h_dst, QwT, Qb, W1p, W2p)


def _c3_body(ctx_ref, w3_ref, c3_ref):
    zc = jnp.maximum(ctx_ref[...], 0.0)
    c3_ref[...] = jnp.dot(zc, w3_ref[...], preferred_element_type=jnp.float32)


def _edge_c3(context_node, W3p):
    BE = 2000
    return pl.pallas_call(
        _c3_body,
        grid=(N_EDGES // BE,),
        in_specs=[
            pl.BlockSpec((BE, DIM), lambda i: (i, 0)),
            pl.BlockSpec((DIM, 8), lambda i: (0, 0)),
        ],
        out_specs=pl.BlockSpec((BE, 8), lambda i: (i, 0)),
        out_shape=jax.ShapeDtypeStruct((N_EDGES, 8), jnp.float32),
    )(context_node, W3p)


def _fin_body(ns_ref, ssum_ref, h_dst_ref, wn_ref, wt_ref, wb_ref, out_ref):
    inv_e = 1.0 / N_EDGES
    nb = ns_ref[...] * inv_e
    tb = (h_dst_ref[...] * ssum_ref[...]) * inv_e
    z = jnp.dot(nb, wn_ref[...], preferred_element_type=jnp.float32)
    z += jnp.dot(tb, wt_ref[...], preferred_element_type=jnp.float32)
    z = jnp.maximum(z + wb_ref[...], 0.0)
    nrm = jnp.sqrt(jnp.sum(z * z, axis=1, keepdims=True))
    nrm = jnp.where(nrm == 0.0, 1.0, nrm)
    out_ref[...] = z / nrm


def _finalize(ns, ssum, h_dst, WnT, WtT, Wb):
    BN = 400
    return pl.pallas_call(
        _fin_body,
        grid=(N_NODES // BN,),
        in_specs=[
            pl.BlockSpec((BN, DIM), lambda i: (i, 0)),
            pl.BlockSpec((BN, DIM), lambda i: (i, 0)),
            pl.BlockSpec((BN, DIM), lambda i: (i, 0)),
            pl.BlockSpec((DIM, DIM), lambda i: (0, 0)),
            pl.BlockSpec((DIM, DIM), lambda i: (0, 0)),
            pl.BlockSpec((1, DIM), lambda i: (0, 0)),
        ],
        out_specs=pl.BlockSpec((BN, DIM), lambda i: (i, 0)),
        out_shape=jax.ShapeDtypeStruct((N_NODES, DIM), jnp.float32),
    )(ns, ssum, h_dst, WnT, WtT, Wb)


def kernel(h_src, h_dst, context_node, edge_index, Q_w, Q_b, W_w, W_b, attn_w):
    src = edge_index[0]
    dst = edge_index[1]
    aw = attn_w[:, 0, :]
    W1 = aw[:, 0:DIM].T
    W2 = aw[:, DIM:2 * DIM].T
    W3 = aw[:, 2 * DIM:3 * DIM].T
    pad = jnp.zeros((DIM, 8 - HEADS), jnp.float32)
    W1p = jnp.concatenate([W1, pad], axis=1)
    W2p = jnp.concatenate([W2, pad], axis=1)
    W3p = jnp.concatenate([W3, pad], axis=1)

    z_src, s1, t2 = _node_precompute(h_src, h_dst, Q_w.T, Q_b[None, :],
                                     W1p, W2p)
    c3 = _edge_c3(context_node, W3p)

    a = s1[src, :HEADS] + t2[dst, :HEADS] + c3[:, :HEADS]
    a = jnp.where(a >= 0, a, 0.01 * a)
    e = jnp.exp(a)
    s = jax.ops.segment_sum(e, dst, num_segments=N_NODES)
    att = jnp.mean(e / s[dst], axis=1, keepdims=True)

    zc = jnp.maximum(context_node, 0.0)
    w = zc * att
    ns = jax.ops.segment_sum(z_src[src] * w, dst, num_segments=N_NODES)
    ssum = jax.ops.segment_sum(w, dst, num_segments=N_NODES)

    return _finalize(ns, ssum, h_dst, W_w.T[:DIM], W_w.T[DIM:], W_b[None, :])


# speedup vs baseline: 11.0625x; 11.0625x over previous
import functools
import jax
import jax.numpy as jnp
from jax import lax
from jax.experimental import pallas as pl
from jax.experimental.pallas import tpu as pltpu
from jax.experimental.pallas import tpu_sc as plsc

N_NODES = 10000
N_EDGES = 320000
DIM = 128
HALF = 64
HEADS = 3
PAD = 8
B = 128
NB_ALL = N_EDGES // B
NROW = N_NODES // 16


def _ab_body(h_src_ref, h_dst_ref, qwt_ref, qb_ref, w1_ref, w2_ref,
             z0_ref, z1_ref, s1_ref, t2_ref):
    z = jnp.maximum(jnp.dot(h_src_ref[...], qwt_ref[...],
                            preferred_element_type=jnp.float32)
                    + qb_ref[...], 0.0)
    z0_ref[...] = z[:, :HALF]
    z1_ref[...] = z[:, HALF:]
    s1_ref[...] = jnp.dot(z, w1_ref[...], preferred_element_type=jnp.float32)
    t2_ref[...] = jnp.dot(h_dst_ref[...], w2_ref[...],
                          preferred_element_type=jnp.float32)


def _node_precompute(h_src, h_dst, QwT, Qb, W1p, W2p):
    BN = 400
    return pl.pallas_call(
        _ab_body,
        grid=(N_NODES // BN,),
        in_specs=[
            pl.BlockSpec((BN, DIM), lambda i: (i, 0)),
            pl.BlockSpec((BN, DIM), lambda i: (i, 0)),
            pl.BlockSpec((DIM, DIM), lambda i: (0, 0)),
            pl.BlockSpec((1, DIM), lambda i: (0, 0)),
            pl.BlockSpec((DIM, PAD), lambda i: (0, 0)),
            pl.BlockSpec((DIM, PAD), lambda i: (0, 0)),
        ],
        out_specs=[
            pl.BlockSpec((BN, HALF), lambda i: (i, 0)),
            pl.BlockSpec((BN, HALF), lambda i: (i, 0)),
            pl.BlockSpec((BN, PAD), lambda i: (i, 0)),
            pl.BlockSpec((BN, PAD), lambda i: (i, 0)),
        ],
        out_shape=[
            jax.ShapeDtypeStruct((N_NODES, HALF), jnp.float32),
            jax.ShapeDtypeStruct((N_NODES, HALF), jnp.float32),
            jax.ShapeDtypeStruct((N_NODES, PAD), jnp.float32),
            jax.ShapeDtypeStruct((N_NODES, PAD), jnp.float32),
        ],
    )(h_src, h_dst, QwT, Qb, W1p, W2p)


def _c3_body(ctx_ref, w3_ref, c3_ref):
    zc = jnp.maximum(ctx_ref[...], 0.0)
    c3_ref[...] = jnp.dot(zc, w3_ref[...], preferred_element_type=jnp.float32)


def _edge_c3(context_node, W3p):
    BE = 2000
    return pl.pallas_call(
        _c3_body,
        grid=(N_EDGES // BE,),
        in_specs=[
            pl.BlockSpec((BE, DIM), lambda i: (i, 0)),
            pl.BlockSpec((DIM, PAD), lambda i: (0, 0)),
        ],
        out_specs=pl.BlockSpec((BE, PAD), lambda i: (i, 0)),
        out_shape=jax.ShapeDtypeStruct((N_EDGES, PAD), jnp.float32),
    )(context_node, W3p)


def _c1_body(src_hbm, dst_hbm, s1_hbm, t2_hbm, c3_hbm, zeros8_hbm,
             e_hbm, sp0_hbm, sp1_hbm,
             srcv, dstv, s1r, t2r, c3r, ebuf, s_sh, sem):
    c = lax.axis_index("c")
    t = lax.axis_index("s")
    wid = t * 2 + c
    pltpu.sync_copy(zeros8_hbm.at[pl.ds(t * NROW, NROW)],
                    s_sh.at[pl.ds(t * NROW, NROW)])
    pltpu.sync_copy(zeros8_hbm.at[pl.ds(0, B)], ebuf)
    plsc.subcore_barrier()

    iota = lax.iota(jnp.int32, 16)
    nb = jnp.where(wid < NB_ALL % 32, NB_ALL // 32 + 1, NB_ALL // 32)

    def batch(b, carry):
        base = (b * 32 + wid) * B
        pltpu.sync_copy(src_hbm.at[pl.ds(base, B)], srcv)
        pltpu.sync_copy(dst_hbm.at[pl.ds(base, B)], dstv)
        pltpu.async_copy(s1_hbm.at[srcv], s1r, sem).wait()
        pltpu.async_copy(t2_hbm.at[dstv], t2r, sem).wait()
        pltpu.sync_copy(c3_hbm.at[pl.ds(base, B)], c3r)

        def group(g, carry2):
            rows = g * 16 + iota
            for h in range(HEADS):
                cols = jnp.full((16,), h, jnp.int32)
                a = (plsc.load_gather(s1r, [rows, cols])
                     + plsc.load_gather(t2r, [rows, cols])
                     + plsc.load_gather(c3r, [rows, cols]))
                a = jnp.maximum(a, a * 0.01)
                plsc.store_scatter(ebuf, [rows, cols], jnp.exp(a))
            return carry2

        lax.fori_loop(0, B // 16, group, 0)
        pltpu.sync_copy(ebuf, e_hbm.at[pl.ds(base, B)])
        pltpu.sync_copy(ebuf, s_sh.at[dstv], add=True)
        return carry

    lax.fori_loop(0, nb, batch, 0)
    plsc.subcore_barrier()

    @pl.when(c == 0)
    def _():
        pltpu.sync_copy(s_sh.at[pl.ds(t * NROW, NROW)],
                        sp0_hbm.at[pl.ds(t * NROW, NROW)])

    @pl.when(c == 1)
    def _():
        pltpu.sync_copy(s_sh.at[pl.ds(t * NROW, NROW)],
                        sp1_hbm.at[pl.ds(t * NROW, NROW)])


def _c1(src, dst, s1, t2, c3, zeros8):
    mesh = plsc.VectorSubcoreMesh(core_axis_name="c", subcore_axis_name="s")
    f = pl.kernel(
        _c1_body,
        out_type=[
            jax.ShapeDtypeStruct((N_EDGES, PAD), jnp.float32),
            jax.ShapeDtypeStruct((N_NODES, PAD), jnp.float32),
            jax.ShapeDtypeStruct((N_NODES, PAD), jnp.float32),
        ],
        mesh=mesh,
        scratch_types=[
            pltpu.VMEM((B,), jnp.int32),
            pltpu.VMEM((B,), jnp.int32),
            pltpu.VMEM((B, PAD), jnp.float32),
            pltpu.VMEM((B, PAD), jnp.float32),
            pltpu.VMEM((B, PAD), jnp.float32),
            pltpu.VMEM((B, PAD), jnp.float32),
            pltpu.VMEM_SHARED((N_NODES, PAD), jnp.float32),
            pltpu.SemaphoreType.DMA,
        ],
        compiler_params=pltpu.CompilerParams(use_tc_tiling_on_sc=False, needs_layout_passes=False),
    )
    return f(src, dst, s1, t2, c3, zeros8)


def _c2_body(dst_hbm, e_hbm, sp0_hbm, sp1_hbm, att_hbm,
             dstv, er, r0, r1, attv, sem):
    c = lax.axis_index("c")
    t = lax.axis_index("s")
    wid = t * 2 + c
    iota = lax.iota(jnp.int32, 16)
    nb = jnp.where(wid < NB_ALL % 32, NB_ALL // 32 + 1, NB_ALL // 32)

    def batch(b, carry):
        base = (b * 32 + wid) * B
        pltpu.sync_copy(dst_hbm.at[pl.ds(base, B)], dstv)
        pltpu.async_copy(sp0_hbm.at[dstv], r0, sem).wait()
        pltpu.async_copy(sp1_hbm.at[dstv], r1, sem).wait()
        pltpu.sync_copy(e_hbm.at[pl.ds(base, B)], er)

        def group(g, carry2):
            rows = g * 16 + iota
            acc = jnp.zeros((16,), jnp.float32)
            for h in range(HEADS):
                cols = jnp.full((16,), h, jnp.int32)
                sv = (plsc.load_gather(r0, [rows, cols])
                      + plsc.load_gather(r1, [rows, cols]))
                acc = acc + plsc.load_gather(er, [rows, cols]) / sv
            attv[pl.ds(g * 16, 16)] = acc * (1.0 / HEADS)
            return carry2

        lax.fori_loop(0, B // 16, group, 0)
        pltpu.sync_copy(attv, att_hbm.at[pl.ds(base, B)])
        return carry

    lax.fori_loop(0, nb, batch, 0)


def _c2(dst, e_arr, sp0, sp1):
    mesh = plsc.VectorSubcoreMesh(core_axis_name="c", subcore_axis_name="s")
    f = pl.kernel(
        _c2_body,
        out_type=jax.ShapeDtypeStruct((N_EDGES,), jnp.float32),
        mesh=mesh,
        scratch_types=[
            pltpu.VMEM((B,), jnp.int32),
            pltpu.VMEM((B, PAD), jnp.float32),
            pltpu.VMEM((B, PAD), jnp.float32),
            pltpu.VMEM((B, PAD), jnp.float32),
            pltpu.VMEM((B,), jnp.float32),
            pltpu.SemaphoreType.DMA,
        ],
        compiler_params=pltpu.CompilerParams(use_tc_tiling_on_sc=False, needs_layout_passes=False),
    )
    return f(dst, e_arr, sp0, sp1)


def _d_body(src_hbm, dst_hbm, att_hbm, ctx_hbm, z0_hbm, z1_hbm, zeros128_hbm,
            out_hbm, srcv, dstv, attv, ctxv, zrows, buf, acc_sh, sem):
    c = lax.axis_index("c")
    t = lax.axis_index("s")
    pltpu.sync_copy(zeros128_hbm.at[pl.ds(t * NROW, NROW)],
                    acc_sh.at[pl.ds(t * NROW, NROW)])
    plsc.subcore_barrier()

    nb = jnp.where(t < NB_ALL % 16, NB_ALL // 16 + 1, NB_ALL // 16)

    def batch(b, carry):
        base = (b * 16 + t) * B
        pltpu.sync_copy(src_hbm.at[pl.ds(base, B)], srcv)
        pltpu.sync_copy(dst_hbm.at[pl.ds(base, B)], dstv)
        pltpu.sync_copy(att_hbm.at[pl.ds(base, B)], attv)

        @pl.when(c == 0)
        def _():
            pltpu.sync_copy(ctx_hbm.at[pl.ds(base, B), pl.ds(0, HALF)], ctxv)
            pltpu.async_copy(z0_hbm.at[srcv], zrows, sem).wait()

        @pl.when(c == 1)
        def _():
            pltpu.sync_copy(ctx_hbm.at[pl.ds(base, B), pl.ds(HALF, HALF)],
                            ctxv)
            pltpu.async_copy(z1_hbm.at[srcv], zrows, sem).wait()

        def edge(e, carry2):
            ab = plsc.load_gather(attv, [jnp.full((16,), e, jnp.int32)])
            for j in range(HALF // 16):
                w = jnp.maximum(ctxv[e, pl.ds(j * 16, 16)], 0.0) * ab
                buf[e, pl.ds(HALF + j * 16, 16)] = w
                buf[e, pl.ds(j * 16, 16)] = w * zrows[e, pl.ds(j * 16, 16)]
            return carry2

        lax.fori_loop(0, B, edge, 0)
        pltpu.sync_copy(buf, acc_sh.at[dstv], add=True)
        return carry

    lax.fori_loop(0, nb, batch, 0)
    plsc.subcore_barrier()
    pltpu.sync_copy(acc_sh.at[pl.ds(t * NROW, NROW)],
                    out_hbm.at[pl.ds(c * N_NODES + t * NROW, NROW)])


def _d(src, dst, att, context_node, z0, z1, zeros128):
    mesh = plsc.VectorSubcoreMesh(core_axis_name="c", subcore_axis_name="s")
    f = pl.kernel(
        _d_body,
        out_type=jax.ShapeDtypeStruct((2 * N_NODES, DIM), jnp.float32),
        mesh=mesh,
        scratch_types=[
            pltpu.VMEM((B,), jnp.int32),
            pltpu.VMEM((B,), jnp.int32),
            pltpu.VMEM((B,), jnp.float32),
            pltpu.VMEM((B, HALF), jnp.float32),
            pltpu.VMEM((B, HALF), jnp.float32),
            pltpu.VMEM((B, DIM), jnp.float32),
            pltpu.VMEM_SHARED((N_NODES, DIM), jnp.float32),
            pltpu.SemaphoreType.DMA,
        ],
        compiler_params=pltpu.CompilerParams(use_tc_tiling_on_sc=False, needs_layout_passes=False),
    )
    return f(src, dst, att, context_node, z0, z1, zeros128)


def _fin_body(ns_ref, ssum_ref, h_dst_ref, wn_ref, wt_ref, wb_ref, out_ref):
    inv_e = 1.0 / N_EDGES
    nb = ns_ref[...] * inv_e
    tb = (h_dst_ref[...] * ssum_ref[...]) * inv_e
    z = jnp.dot(nb, wn_ref[...], preferred_element_type=jnp.float32)
    z += jnp.dot(tb, wt_ref[...], preferred_element_type=jnp.float32)
    z = jnp.maximum(z + wb_ref[...], 0.0)
    nrm = jnp.sqrt(jnp.sum(z * z, axis=1, keepdims=True))
    nrm = jnp.where(nrm == 0.0, 1.0, nrm)
    out_ref[...] = z / nrm


def _finalize(ns, ssum, h_dst, WnT, WtT, Wb):
    BN = 400
    return pl.pallas_call(
        _fin_body,
        grid=(N_NODES // BN,),
        in_specs=[
            pl.BlockSpec((BN, DIM), lambda i: (i, 0)),
            pl.BlockSpec((BN, DIM), lambda i: (i, 0)),
            pl.BlockSpec((BN, DIM), lambda i: (i, 0)),
            pl.BlockSpec((DIM, DIM), lambda i: (0, 0)),
            pl.BlockSpec((DIM, DIM), lambda i: (0, 0)),
            pl.BlockSpec((1, DIM), lambda i: (0, 0)),
        ],
        out_specs=pl.BlockSpec((BN, DIM), lambda i: (i, 0)),
        out_shape=jax.ShapeDtypeStruct((N_NODES, DIM), jnp.float32),
    )(ns, ssum, h_dst, WnT, WtT, Wb)


def kernel(h_src, h_dst, context_node, edge_index, Q_w, Q_b, W_w, W_b, attn_w):
    src = edge_index[0]
    dst = edge_index[1]
    aw = attn_w[:, 0, :]
    W1 = aw[:, 0:DIM].T
    W2 = aw[:, DIM:2 * DIM].T
    W3 = aw[:, 2 * DIM:3 * DIM].T
    pad = jnp.zeros((DIM, PAD - HEADS), jnp.float32)
    W1p = jnp.concatenate([W1, pad], axis=1)
    W2p = jnp.concatenate([W2, pad], axis=1)
    W3p = jnp.concatenate([W3, pad], axis=1)
    zeros8 = jnp.zeros((N_NODES, PAD), jnp.float32)
    zeros128 = jnp.zeros((N_NODES, DIM), jnp.float32)

    z0, z1, s1, t2 = _node_precompute(h_src, h_dst, Q_w.T, Q_b[None, :],
                                      W1p, W2p)
    c3 = _edge_c3(context_node, W3p)
    e_arr, sp0, sp1 = _c1(src, dst, s1, t2, c3, zeros8)
    att = _c2(dst, e_arr, sp0, sp1)
    d_out = _d(src, dst, att, context_node, z0, z1, zeros128)
    ns = jnp.concatenate([d_out[:N_NODES, :HALF], d_out[N_NODES:, :HALF]],
                         axis=1)
    ssum = jnp.concatenate([d_out[:N_NODES, HALF:], d_out[N_NODES:, HALF:]],
                           axis=1)
    return _finalize(ns, ssum, h_dst, W_w.T[:DIM], W_w.T[DIM:], W_b[None, :])
